# baseline (speedup 1.0000x reference)
import functools

import jax
import jax.numpy as jnp
from jax import lax
from jax.experimental import pallas as pl
from jax.experimental.pallas import tpu as pltpu
from jax.experimental.pallas import tpu_sc as plsc

N = 10000
D = 128
E = 320000
NC = 2
NS = 16
NW = NC * NS
CH = 128
K0 = 112
K1 = 48
TOTCH = NS * (K0 + K1)
E_PAD = TOTCH * CH
NCH_DEG = TOTCH // NW
ACC_ROWS = 10240
ROWS_PER_TILE = ACC_ROWS // NS
JUNK_ROW = 10200


def _deg_body(dst_hbm, zeros_hbm, out_hbm, idx_d, acc):
    c = lax.axis_index("c")
    s = lax.axis_index("s")
    wid = c * NS + s
    pltpu.sync_copy(dst_hbm.at[wid], idx_d)
    pltpu.sync_copy(zeros_hbm, acc)
    ones16 = jnp.ones((16,), jnp.float32)

    def body(j, carry):
        def inner(k, carry2):
            idx = idx_d[j, pl.ds(k * 16, 16)]
            plsc.addupdate_scatter(acc, [idx], ones16)
            return carry2
        return lax.fori_loop(0, CH // 16, inner, carry)

    lax.fori_loop(0, NCH_DEG, body, 0)
    pltpu.sync_copy(acc, out_hbm.at[wid])


def _agg_body(eidx_hbm, srcf_hbm, dstf_hbm, p_hbm, zeros_hbm, out_hbm,
              ia, ib, bs, bd, rows, acc, isa, isb, gsa, gsb):
    c = lax.axis_index("c")
    s = lax.axis_index("s")
    rowbase = c * (NS * K0) + s * (K0 + c * (K1 - K0))

    ra = rows.at[0]
    rb = rows.at[1]

    def stage(j, ibuf, isem):
        pltpu.async_copy(eidx_hbm.at[rowbase + j], ibuf, isem)

    def wait_stage(ibuf, isem):
        pltpu.make_async_copy(eidx_hbm.at[0], ibuf, isem).wait()

    def gather(ibuf, rbuf, gsem):
        pltpu.async_copy(p_hbm.at[ibuf.at[0]], rbuf, gsem)

    def wait_g(rbuf, gsem):
        pltpu.make_async_copy(p_hbm.at[ia.at[0]], rbuf, gsem).wait()

    def scat(ibuf, rbuf):
        pltpu.sync_copy(rbuf, acc.at[ibuf.at[1]], add=True)

    def zbody(k, carry):
        pltpu.sync_copy(
            zeros_hbm, acc.at[pl.ds(s * ROWS_PER_TILE + k * 128, 128)])
        return carry

    lax.fori_loop(0, ROWS_PER_TILE // 128, zbody, 0)
    plsc.subcore_barrier()

    @pl.when(c == 0)
    def _pipelined():
        stage(0, ia, isa)
        wait_stage(ia, isa)
        gather(ia, ra, gsa)
        stage(1, ib, isb)

        def pair(jj, carry):
            j = 2 * jj
            wait_g(ra, gsa)
            wait_stage(ib, isb)
            gather(ib, rb, gsb)
            scat(ia, ra)
            stage(j + 2, ia, isa)
            wait_g(rb, gsb)
            wait_stage(ia, isa)
            gather(ia, ra, gsa)
            scat(ib, rb)
            stage(j + 3, ib, isb)
            return carry

        lax.fori_loop(0, K0 // 2 - 1, pair, 0)
        wait_g(ra, gsa)
        wait_stage(ib, isb)
        gather(ib, rb, gsb)
        scat(ia, ra)
        wait_g(rb, gsb)
        scat(ib, rb)

    @pl.when(c == 1)
    def _serial():
        pltpu.async_copy(srcf_hbm.at[pl.ds(rowbase, K1)], bs, isa)
        pltpu.async_copy(dstf_hbm.at[pl.ds(rowbase, K1)], bd, isa)
        pltpu.make_async_copy(srcf_hbm.at[pl.ds(0, K1)], bs, isa).wait()
        pltpu.make_async_copy(dstf_hbm.at[pl.ds(0, K1)], bd, isa).wait()

        def body(j, carry):
            pltpu.async_copy(p_hbm.at[bs.at[j]], ra, gsa)
            pltpu.make_async_copy(p_hbm.at[bs.at[j]], ra, gsa).wait()
            pltpu.sync_copy(ra, acc.at[bd.at[j]], add=True)
            return carry

        lax.fori_loop(0, K1, body, 0)

    plsc.subcore_barrier()
    pltpu.sync_copy(
        acc.at[pl.ds(s * ROWS_PER_TILE, ROWS_PER_TILE)],
        out_hbm.at[c, pl.ds(s * ROWS_PER_TILE, ROWS_PER_TILE)],
    )


@functools.lru_cache(maxsize=None)
def _sc_kernels():
    mesh = plsc.VectorSubcoreMesh(core_axis_name="c", subcore_axis_name="s")
    deg_sc = pl.kernel(
        _deg_body,
        out_type=jax.ShapeDtypeStruct((NW, ACC_ROWS), jnp.float32),
        mesh=mesh,
        scratch_types=[
            pltpu.VMEM((NCH_DEG, CH), jnp.int32),
            pltpu.VMEM((ACC_ROWS,), jnp.float32),
        ],
        compiler_params=pltpu.CompilerParams(needs_layout_passes=False),
    )
    agg_sc = pl.kernel(
        _agg_body,
        out_type=jax.ShapeDtypeStruct((NC, ACC_ROWS, D), jnp.float32),
        mesh=mesh,
        scratch_types=[
            pltpu.VMEM((2, CH), jnp.int32),
            pltpu.VMEM((2, CH), jnp.int32),
            pltpu.VMEM((K1, CH), jnp.int32),
            pltpu.VMEM((K1, CH), jnp.int32),
            pltpu.VMEM((2, CH, D), jnp.float32),
            pltpu.VMEM_SHARED((ACC_ROWS, D), jnp.float32),
            pltpu.SemaphoreType.DMA,
            pltpu.SemaphoreType.DMA,
            pltpu.SemaphoreType.DMA,
            pltpu.SemaphoreType.DMA,
        ],
    )
    return deg_sc, agg_sc


_BM = 1000
_NB = N // _BM
_BD = 1280
_NBD = ACC_ROWS // _BD


def _mm_body(x_ref, w_ref, o_ref):
    o_ref[...] = jnp.dot(x_ref[...], w_ref[...],
                         preferred_element_type=jnp.float32)


def _matmul_tc(x, w):
    return pl.pallas_call(
        _mm_body,
        grid=(_NB,),
        in_specs=[
            pl.BlockSpec((_BM, D), lambda i: (i, 0)),
            pl.BlockSpec((D, D), lambda i: (0, 0)),
        ],
        out_specs=pl.BlockSpec((_BM, D), lambda i: (i, 0)),
        out_shape=jax.ShapeDtypeStruct((N, D), jnp.float32),
    )(x, w)


def _dinv_body(degpt_ref, o_ref):
    deg = jnp.sum(degpt_ref[...], axis=1, keepdims=True) + 1.0
    o_ref[...] = jnp.broadcast_to(lax.rsqrt(deg), (_BD, D))


def _dinv_tc(degpt):
    return pl.pallas_call(
        _dinv_body,
        grid=(_NBD,),
        in_specs=[pl.BlockSpec((_BD, NW), lambda i: (i, 0))],
        out_specs=pl.BlockSpec((_BD, D), lambda i: (i, 0)),
        out_shape=jax.ShapeDtypeStruct((ACC_ROWS, D), jnp.float32),
    )(degpt)


def _mul_body(h_ref, dinv_ref, o_ref):
    o_ref[...] = h_ref[...] * dinv_ref[...]


def _mul_tc(h, dinvf):
    return pl.pallas_call(
        _mul_body,
        grid=(_NB,),
        in_specs=[
            pl.BlockSpec((_BM, D), lambda i: (i, 0)),
            pl.BlockSpec((_BM, D), lambda i: (i, 0)),
        ],
        out_specs=pl.BlockSpec((_BM, D), lambda i: (i, 0)),
        out_shape=jax.ShapeDtypeStruct((N, D), jnp.float32),
    )(h, dinvf)


def _mid_body(a_ref, p_ref, dinv_ref, b_ref, w_ref, o_ref):
    t = dinv_ref[...] * (a_ref[0] + a_ref[1] + p_ref[...]) + b_ref[...]
    t = jnp.maximum(t, 0.0)
    o_ref[...] = jnp.dot(t, w_ref[...],
                         preferred_element_type=jnp.float32) * dinv_ref[...]


def _mid_tc(aggp, p, dinvf, b, w):
    return pl.pallas_call(
        _mid_body,
        grid=(_NB,),
        in_specs=[
            pl.BlockSpec((NC, _BM, D), lambda i: (0, i, 0)),
            pl.BlockSpec((_BM, D), lambda i: (i, 0)),
            pl.BlockSpec((_BM, D), lambda i: (i, 0)),
            pl.BlockSpec((1, D), lambda i: (0, 0)),
            pl.BlockSpec((D, D), lambda i: (0, 0)),
        ],
        out_specs=pl.BlockSpec((_BM, D), lambda i: (i, 0)),
        out_shape=jax.ShapeDtypeStruct((N, D), jnp.float32),
    )(aggp, p, dinvf, b, w)


def _fin_body(a_ref, p_ref, dinv_ref, b_ref, o_ref):
    t = dinv_ref[...] * (a_ref[0] + a_ref[1] + p_ref[...]) + b_ref[...]
    o_ref[...] = jax.nn.sigmoid(t)


def _fin_tc(aggp, p, dinvf, b):
    return pl.pallas_call(
        _fin_body,
        grid=(_NB,),
        in_specs=[
            pl.BlockSpec((NC, _BM, D), lambda i: (0, i, 0)),
            pl.BlockSpec((_BM, D), lambda i: (i, 0)),
            pl.BlockSpec((_BM, D), lambda i: (i, 0)),
            pl.BlockSpec((1, D), lambda i: (0, 0)),
        ],
        out_specs=pl.BlockSpec((_BM, D), lambda i: (i, 0)),
        out_shape=jax.ShapeDtypeStruct((N, D), jnp.float32),
    )(aggp, p, dinvf, b)


def kernel(x, edge_index, W1, b1, W2, b2):
    ei = edge_index.astype(jnp.int32)
    pad = E_PAD - E
    src = jnp.concatenate([ei[0], jnp.zeros((pad,), jnp.int32)])
    dst = jnp.concatenate([ei[1], jnp.full((pad,), JUNK_ROW, jnp.int32)])
    srcf = src.reshape(TOTCH, CH)
    dstf = dst.reshape(TOTCH, CH)
    eidx = jnp.stack([srcf, dstf], axis=1)
    dst_r = dst.reshape(NW, NCH_DEG, CH)

    zeros_deg = jnp.zeros((ACC_ROWS,), jnp.float32)
    zeros_rows = jnp.zeros((128, D), jnp.float32)

    deg_sc, agg_sc = _sc_kernels()
    degp = deg_sc(dst_r, zeros_deg)
    h1 = _matmul_tc(x, W1)
    dinvf = _dinv_tc(degp.T)
    p1 = _mul_tc(h1, dinvf)

    agg1 = agg_sc(eidx, srcf, dstf, p1, zeros_rows)
    p2 = _mid_tc(agg1, p1, dinvf, b1[None, :], W2)

    agg2 = agg_sc(eidx, srcf, dstf, p2, zeros_rows)
    return _fin_tc(agg2, p2, dinvf, b2[None, :])


# speedup vs baseline: 1.1600x; 1.1600x over previous
import functools

import jax
import jax.numpy as jnp
from jax import lax
from jax.experimental import pallas as pl
from jax.experimental.pallas import tpu as pltpu
from jax.experimental.pallas import tpu_sc as plsc

N = 10000
D = 128
E = 320000
NC = 2
NS = 16
NW = NC * NS
CH = 128
K0 = 132
K1 = 28
TOTCH = NS * (K0 + K1)
E_PAD = TOTCH * CH
NCH_DEG = TOTCH // NW
ACC_ROWS = 10240
ROWS_PER_TILE = ACC_ROWS // NS
JUNK_ROW = 10200


def _deg_body(dst_hbm, zeros_hbm, out_hbm, idx_d, acc):
    c = lax.axis_index("c")
    s = lax.axis_index("s")
    wid = c * NS + s
    pltpu.sync_copy(dst_hbm.at[wid], idx_d)
    pltpu.sync_copy(zeros_hbm, acc)
    ones16 = jnp.ones((16,), jnp.float32)

    def body(j, carry):
        def inner(k, carry2):
            idx = idx_d[j, pl.ds(k * 16, 16)]
            plsc.addupdate_scatter(acc, [idx], ones16)
            return carry2
        return lax.fori_loop(0, CH // 16, inner, carry)

    lax.fori_loop(0, NCH_DEG, body, 0)
    pltpu.sync_copy(acc, out_hbm.at[wid])


def _agg_body(eidx_hbm, srcf_hbm, dstf_hbm, p_hbm, zeros_hbm, out_hbm,
              ia, ib, bs, bd, rows, acc, isa, isb, gsa, gsb):
    c = lax.axis_index("c")
    s = lax.axis_index("s")
    rowbase = c * (NS * K0) + s * (K0 + c * (K1 - K0))

    ra = rows.at[0]
    rb = rows.at[1]

    def stage(j, ibuf, isem):
        pltpu.async_copy(eidx_hbm.at[rowbase + j], ibuf, isem)

    def wait_stage(ibuf, isem):
        pltpu.make_async_copy(eidx_hbm.at[0], ibuf, isem).wait()

    def gather(ibuf, rbuf, gsem):
        pltpu.async_copy(p_hbm.at[ibuf.at[0]], rbuf, gsem)

    def wait_g(rbuf, gsem):
        pltpu.make_async_copy(p_hbm.at[ia.at[0]], rbuf, gsem).wait()

    def scat(ibuf, rbuf):
        pltpu.sync_copy(rbuf, acc.at[ibuf.at[1]], add=True)

    def zbody(k, carry):
        pltpu.sync_copy(
            zeros_hbm, acc.at[pl.ds(s * ROWS_PER_TILE + k * 128, 128)])
        return carry

    lax.fori_loop(0, ROWS_PER_TILE // 128, zbody, 0)
    plsc.subcore_barrier()

    @pl.when(c == 0)
    def _pipelined():
        stage(0, ia, isa)
        wait_stage(ia, isa)
        gather(ia, ra, gsa)
        stage(1, ib, isb)

        def pair(jj, carry):
            j = 2 * jj
            wait_g(ra, gsa)
            wait_stage(ib, isb)
            gather(ib, rb, gsb)
            scat(ia, ra)
            stage(j + 2, ia, isa)
            wait_g(rb, gsb)
            wait_stage(ia, isa)
            gather(ia, ra, gsa)
            scat(ib, rb)
            stage(j + 3, ib, isb)
            return carry

        lax.fori_loop(0, K0 // 2 - 1, pair, 0)
        wait_g(ra, gsa)
        wait_stage(ib, isb)
        gather(ib, rb, gsb)
        scat(ia, ra)
        wait_g(rb, gsb)
        scat(ib, rb)

    @pl.when(c == 1)
    def _serial():
        pltpu.async_copy(srcf_hbm.at[s], bs, isa)
        pltpu.async_copy(dstf_hbm.at[s], bd, isa)
        pltpu.make_async_copy(srcf_hbm.at[0], bs, isa).wait()
        pltpu.make_async_copy(dstf_hbm.at[0], bd, isa).wait()

        def body(j, carry):
            pltpu.async_copy(p_hbm.at[bs.at[j]], ra, gsa)
            pltpu.make_async_copy(p_hbm.at[bs.at[j]], ra, gsa).wait()
            pltpu.sync_copy(ra, acc.at[bd.at[j]], add=True)
            return carry

        lax.fori_loop(0, K1, body, 0)

    plsc.subcore_barrier()
    pltpu.sync_copy(
        acc.at[pl.ds(s * ROWS_PER_TILE, ROWS_PER_TILE)],
        out_hbm.at[c, pl.ds(s * ROWS_PER_TILE, ROWS_PER_TILE)],
    )


@functools.lru_cache(maxsize=None)
def _sc_kernels():
    mesh = plsc.VectorSubcoreMesh(core_axis_name="c", subcore_axis_name="s")
    deg_sc = pl.kernel(
        _deg_body,
        out_type=jax.ShapeDtypeStruct((NW, ACC_ROWS), jnp.float32),
        mesh=mesh,
        scratch_types=[
            pltpu.VMEM((NCH_DEG, CH), jnp.int32),
            pltpu.VMEM((ACC_ROWS,), jnp.float32),
        ],
        compiler_params=pltpu.CompilerParams(needs_layout_passes=False),
    )
    agg_sc = pl.kernel(
        _agg_body,
        out_type=jax.ShapeDtypeStruct((NC, ACC_ROWS, D), jnp.float32),
        mesh=mesh,
        scratch_types=[
            pltpu.VMEM((2, CH), jnp.int32),
            pltpu.VMEM((2, CH), jnp.int32),
            pltpu.VMEM((K1, CH), jnp.int32),
            pltpu.VMEM((K1, CH), jnp.int32),
            pltpu.VMEM((2, CH, D), jnp.float32),
            pltpu.VMEM_SHARED((ACC_ROWS, D), jnp.float32),
            pltpu.SemaphoreType.DMA,
            pltpu.SemaphoreType.DMA,
            pltpu.SemaphoreType.DMA,
            pltpu.SemaphoreType.DMA,
        ],
    )
    return deg_sc, agg_sc


_BM = 1000
_NB = N // _BM
_BD = 1280
_NBD = ACC_ROWS // _BD


def _mm_body(x_ref, w_ref, o_ref):
    o_ref[...] = jnp.dot(x_ref[...], w_ref[...],
                         preferred_element_type=jnp.float32)


def _matmul_tc(x, w):
    return pl.pallas_call(
        _mm_body,
        grid=(_NB,),
        in_specs=[
            pl.BlockSpec((_BM, D), lambda i: (i, 0)),
            pl.BlockSpec((D, D), lambda i: (0, 0)),
        ],
        out_specs=pl.BlockSpec((_BM, D), lambda i: (i, 0)),
        out_shape=jax.ShapeDtypeStruct((N, D), jnp.float32),
    )(x, w)


def _dinv_body(degpt_ref, o_ref):
    deg = jnp.sum(degpt_ref[...], axis=1, keepdims=True) + 1.0
    o_ref[...] = jnp.broadcast_to(lax.rsqrt(deg), (_BD, D))


def _dinv_tc(degpt):
    return pl.pallas_call(
        _dinv_body,
        grid=(_NBD,),
        in_specs=[pl.BlockSpec((_BD, NW), lambda i: (i, 0))],
        out_specs=pl.BlockSpec((_BD, D), lambda i: (i, 0)),
        out_shape=jax.ShapeDtypeStruct((ACC_ROWS, D), jnp.float32),
    )(degpt)


def _mul_body(h_ref, dinv_ref, o_ref):
    o_ref[...] = h_ref[...] * dinv_ref[...]


def _mul_tc(h, dinvf):
    return pl.pallas_call(
        _mul_body,
        grid=(_NB,),
        in_specs=[
            pl.BlockSpec((_BM, D), lambda i: (i, 0)),
            pl.BlockSpec((_BM, D), lambda i: (i, 0)),
        ],
        out_specs=pl.BlockSpec((_BM, D), lambda i: (i, 0)),
        out_shape=jax.ShapeDtypeStruct((N, D), jnp.float32),
    )(h, dinvf)


def _mid_body(a_ref, p_ref, dinv_ref, b_ref, w_ref, o_ref):
    t = dinv_ref[...] * (a_ref[0] + a_ref[1] + p_ref[...]) + b_ref[...]
    t = jnp.maximum(t, 0.0)
    o_ref[...] = jnp.dot(t, w_ref[...],
                         preferred_element_type=jnp.float32) * dinv_ref[...]


def _mid_tc(aggp, p, dinvf, b, w):
    return pl.pallas_call(
        _mid_body,
        grid=(_NB,),
        in_specs=[
            pl.BlockSpec((NC, _BM, D), lambda i: (0, i, 0)),
            pl.BlockSpec((_BM, D), lambda i: (i, 0)),
            pl.BlockSpec((_BM, D), lambda i: (i, 0)),
            pl.BlockSpec((1, D), lambda i: (0, 0)),
            pl.BlockSpec((D, D), lambda i: (0, 0)),
        ],
        out_specs=pl.BlockSpec((_BM, D), lambda i: (i, 0)),
        out_shape=jax.ShapeDtypeStruct((N, D), jnp.float32),
    )(aggp, p, dinvf, b, w)


def _fin_body(a_ref, p_ref, dinv_ref, b_ref, o_ref):
    t = dinv_ref[...] * (a_ref[0] + a_ref[1] + p_ref[...]) + b_ref[...]
    o_ref[...] = jax.nn.sigmoid(t)


def _fin_tc(aggp, p, dinvf, b):
    return pl.pallas_call(
        _fin_body,
        grid=(_NB,),
        in_specs=[
            pl.BlockSpec((NC, _BM, D), lambda i: (0, i, 0)),
            pl.BlockSpec((_BM, D), lambda i: (i, 0)),
            pl.BlockSpec((_BM, D), lambda i: (i, 0)),
            pl.BlockSpec((1, D), lambda i: (0, 0)),
        ],
        out_specs=pl.BlockSpec((_BM, D), lambda i: (i, 0)),
        out_shape=jax.ShapeDtypeStruct((N, D), jnp.float32),
    )(aggp, p, dinvf, b)


def kernel(x, edge_index, W1, b1, W2, b2):
    ei = edge_index.astype(jnp.int32)
    pad = E_PAD - E
    src = jnp.concatenate([ei[0], jnp.zeros((pad,), jnp.int32)])
    dst = jnp.concatenate([ei[1], jnp.full((pad,), JUNK_ROW, jnp.int32)])
    srcf = src.reshape(TOTCH, CH)
    dstf = dst.reshape(TOTCH, CH)
    eidx = jnp.stack([srcf, dstf], axis=1)
    srcf1 = srcf[NS * K0:].reshape(NS, K1, CH)
    dstf1 = dstf[NS * K0:].reshape(NS, K1, CH)
    dst_r = dst.reshape(NW, NCH_DEG, CH)

    zeros_deg = jnp.zeros((ACC_ROWS,), jnp.float32)
    zeros_rows = jnp.zeros((128, D), jnp.float32)

    deg_sc, agg_sc = _sc_kernels()
    degp = deg_sc(dst_r, zeros_deg)
    h1 = _matmul_tc(x, W1)
    dinvf = _dinv_tc(degp.T)
    p1 = _mul_tc(h1, dinvf)

    agg1 = agg_sc(eidx, srcf1, dstf1, p1, zeros_rows)
    p2 = _mid_tc(agg1, p1, dinvf, b1[None, :], W2)

    agg2 = agg_sc(eidx, srcf1, dstf1, p2, zeros_rows)
    return _fin_tc(agg2, p2, dinvf, b2[None, :])
